# baseline (speedup 1.0000x reference)
import jax
import jax.numpy as jnp
from jax.experimental import pallas as pl
from jax.experimental.pallas import tpu as pltpu

U_BLK = 3072


def _moe_body(x_ref, rw_ref, bu_ref, bd_ref, wup_ref, wdn_ref, out_ref,
              gate_ref):
    e = pl.program_id(0)
    u = pl.program_id(1)
    nu = pl.num_programs(1)
    n_e = pl.num_programs(0)

    @pl.when((e == 0) & (u == 0))
    def _init():
        x = x_ref[...]
        rw = rw_ref[...]
        logits = jnp.dot(x, rw.T, preferred_element_type=jnp.float32)
        col = jax.lax.broadcasted_iota(jnp.int32, logits.shape, 1)
        m1 = jnp.max(logits, axis=1, keepdims=True)
        i1 = jnp.min(jnp.where(logits == m1, col, n_e), axis=1, keepdims=True)
        masked = jnp.where(col == i1, -jnp.inf, logits)
        m2 = jnp.max(masked, axis=1, keepdims=True)
        i2 = jnp.min(jnp.where(masked == m2, col, n_e), axis=1, keepdims=True)
        t = jnp.exp(m2 - m1)
        w1 = 1.0 / (1.0 + t)
        w2 = t / (1.0 + t)
        gates = w1 * (col == i1) + w2 * (col == i2)
        gate_ref[...] = gates.T
        out_ref[...] = jnp.zeros_like(out_ref)

    x = x_ref[...]
    g = gate_ref[e, :].reshape(-1, 1)
    wup = wup_ref[0]
    h = jnp.dot(x, wup.T, preferred_element_type=jnp.float32)
    h = h + bu_ref[e, pl.ds(u * U_BLK, U_BLK)]
    h = 0.5 * h * (1.0 + jax.lax.erf(h * 0.7071067811865476))
    h = h * g
    wdn = wdn_ref[0]
    out_ref[...] += jnp.dot(h, wdn.T, preferred_element_type=jnp.float32)

    @pl.when(u == nu - 1)
    def _bias_down():
        out_ref[...] += g * bd_ref[e, :]


def kernel(x, expert_weights_up, expert_weights_down, expert_biases_up,
           expert_biases_down, router_weight):
    B, S, D = x.shape
    E, U, _ = expert_weights_up.shape
    x2d = x.reshape(B * S, D)
    nu = U // U_BLK

    out = pl.pallas_call(
        _moe_body,
        grid=(E, nu),
        in_specs=[
            pl.BlockSpec((B * S, D), lambda e, u: (0, 0)),
            pl.BlockSpec((E, D), lambda e, u: (0, 0)),
            pl.BlockSpec((E, U), lambda e, u: (0, 0)),
            pl.BlockSpec((E, D), lambda e, u: (0, 0)),
            pl.BlockSpec((1, U_BLK, D), lambda e, u: (e, u, 0)),
            pl.BlockSpec((1, D, U_BLK), lambda e, u: (e, 0, u)),
        ],
        out_specs=pl.BlockSpec((B * S, D), lambda e, u: (0, 0)),
        out_shape=jax.ShapeDtypeStruct((B * S, D), jnp.float32),
        scratch_shapes=[pltpu.VMEM((E, B * S), jnp.float32)],
    )(x2d, router_weight, expert_biases_up, expert_biases_down,
      expert_weights_up, expert_weights_down)
    return out.reshape(B, S, D)


# speedup vs baseline: 1.0378x; 1.0378x over previous
import functools

import jax
import jax.numpy as jnp
from jax.experimental import pallas as pl
from jax.experimental.pallas import tpu as pltpu

U_BLK = 1536


def _routing_body(x_ref, rw_ref, gates_ref, idx_ref, *, nu, ns):
    x = x_ref[...]
    rw = rw_ref[...]
    B, E = x.shape[0], rw.shape[0]
    logits = jnp.dot(x, rw.T, preferred_element_type=jnp.float32)
    col = jax.lax.broadcasted_iota(jnp.int32, (B, E), 1)
    m1 = jnp.max(logits, axis=1, keepdims=True)
    i1 = jnp.min(jnp.where(logits == m1, col, E), axis=1, keepdims=True)
    masked = jnp.where(col == i1, -jnp.inf, logits)
    m2 = jnp.max(masked, axis=1, keepdims=True)
    i2 = jnp.min(jnp.where(masked == m2, col, E), axis=1, keepdims=True)
    t = jnp.exp(m2 - m1)
    w1 = 1.0 / (1.0 + t)
    w2 = t / (1.0 + t)
    sel1 = (col == i1)
    sel2 = (col == i2)
    gates = w1 * sel1 + w2 * sel2
    gates_ref[...] = gates.T

    act = jnp.max((sel1 | sel2).astype(jnp.float32), axis=0, keepdims=True)
    lt = (jax.lax.broadcasted_iota(jnp.int32, (E, E), 0) <
          jax.lax.broadcasted_iota(jnp.int32, (E, E), 1)).astype(jnp.float32)
    pos = jnp.dot(act, lt, preferred_element_type=jnp.float32)
    nact = jnp.sum(act, dtype=jnp.float32).astype(jnp.int32)
    pos_c = pos.astype(jnp.int32).T
    act_c = act.astype(jnp.int32).T
    slot = jax.lax.broadcasted_iota(jnp.int32, (E, E), 1)
    e_c = jax.lax.broadcasted_iota(jnp.int32, (E, E), 0)
    take = (pos_c == slot) & (act_c == 1)
    order = jnp.sum(jnp.where(take, e_c, 0), axis=0, keepdims=True)

    s_io = jax.lax.broadcasted_iota(jnp.int32, (1, ns), 1)
    log_nu = max(nu.bit_length() - 1, 0)
    si = s_io >> log_nu
    sj = s_io & (nu - 1)
    ci_c = jnp.minimum(si, nact - 1).T
    slot2 = jax.lax.broadcasted_iota(jnp.int32, (ns, E), 1)
    steps_e = jnp.sum(jnp.where(ci_c == slot2, order, 0), axis=1,
                      keepdims=True).T
    scale = (s_io < nact * nu).astype(jnp.int32)
    sj_eff = jnp.where(scale == 1, sj, nu - 1)
    idx_ref[...] = jnp.concatenate([steps_e, sj_eff, scale], axis=0)


def _moe_body(idx_ref, x_ref, gates_ref, bu_ref, bd_ref, wup_ref, wdn_ref,
              out_ref, *, nu):
    s = pl.program_id(0)
    e = idx_ref[0, s]
    uj = idx_ref[1, s]

    @pl.when(s == 0)
    def _init():
        out_ref[...] = jnp.zeros_like(out_ref)

    x = x_ref[...]
    g = gates_ref[e, :].reshape(-1, 1) * idx_ref[2, s].astype(jnp.float32)
    wup = wup_ref[0]
    h = jnp.dot(x, wup.T, preferred_element_type=jnp.float32)
    h = h + bu_ref[e, pl.ds(uj * U_BLK, U_BLK)]
    h = 0.5 * h * (1.0 + jax.lax.erf(h * 0.7071067811865476))
    h = h * g
    wdn = wdn_ref[0]
    out_ref[...] += jnp.dot(h, wdn.T, preferred_element_type=jnp.float32)

    @pl.when(uj == nu - 1)
    def _bias_down():
        out_ref[...] += g * bd_ref[e, :]


def kernel(x, expert_weights_up, expert_weights_down, expert_biases_up,
           expert_biases_down, router_weight):
    B, S, D = x.shape
    E, U, _ = expert_weights_up.shape
    x2d = x.reshape(B * S, D)
    nu = U // U_BLK
    ns = E * nu

    gates_t, idx = pl.pallas_call(
        functools.partial(_routing_body, nu=nu, ns=ns),
        out_shape=[
            jax.ShapeDtypeStruct((E, B * S), jnp.float32),
            jax.ShapeDtypeStruct((3, ns), jnp.int32),
        ],
    )(x2d, router_weight)

    grid_spec = pltpu.PrefetchScalarGridSpec(
        num_scalar_prefetch=1,
        grid=(ns,),
        in_specs=[
            pl.BlockSpec((B * S, D), lambda s, idx: (0, 0)),
            pl.BlockSpec((E, B * S), lambda s, idx: (0, 0)),
            pl.BlockSpec((E, U), lambda s, idx: (0, 0)),
            pl.BlockSpec((E, D), lambda s, idx: (0, 0)),
            pl.BlockSpec((1, U_BLK, D), lambda s, idx: (idx[0, s], idx[1, s], 0)),
            pl.BlockSpec((1, D, U_BLK), lambda s, idx: (idx[0, s], 0, idx[1, s])),
        ],
        out_specs=pl.BlockSpec((B * S, D), lambda s, idx: (0, 0)),
    )
    out = pl.pallas_call(
        functools.partial(_moe_body, nu=nu),
        grid_spec=grid_spec,
        out_shape=jax.ShapeDtypeStruct((B * S, D), jnp.float32),
    )(idx, x2d, gates_t, expert_biases_up, expert_biases_down,
      expert_weights_up, expert_weights_down)
    return out.reshape(B, S, D)


# speedup vs baseline: 1.1223x; 1.0815x over previous
import jax
import jax.numpy as jnp
from jax.experimental import pallas as pl
from jax.experimental.pallas import tpu as pltpu


def _moe_body(x_ref, rw_ref, bu_ref, bd_ref, wup_hbm, wdn_hbm, out_ref,
              gates_ref, tbl_ref, ubuf, dbuf, sem_u, sem_d):
    x = x_ref[...]
    rw = rw_ref[...]
    B, E = x.shape[0], rw.shape[0]

    logits = jnp.dot(x, rw.T, preferred_element_type=jnp.float32)
    col = jax.lax.broadcasted_iota(jnp.int32, (B, E), 1)
    m1 = jnp.max(logits, axis=1, keepdims=True)
    i1 = jnp.min(jnp.where(logits == m1, col, E), axis=1, keepdims=True)
    masked = jnp.where(col == i1, -jnp.inf, logits)
    m2 = jnp.max(masked, axis=1, keepdims=True)
    i2 = jnp.min(jnp.where(masked == m2, col, E), axis=1, keepdims=True)
    t = jnp.exp(m2 - m1)
    w1 = 1.0 / (1.0 + t)
    w2 = t / (1.0 + t)
    sel1 = (col == i1)
    sel2 = (col == i2)
    gates_ref[...] = (w1 * sel1 + w2 * sel2).T

    act = jnp.max((sel1 | sel2).astype(jnp.float32), axis=0, keepdims=True)
    lt = (jax.lax.broadcasted_iota(jnp.int32, (E, E), 0) <
          jax.lax.broadcasted_iota(jnp.int32, (E, E), 1)).astype(jnp.float32)
    pos = jnp.dot(act, lt, preferred_element_type=jnp.float32)
    nactf = jnp.sum(act, dtype=jnp.float32)
    pos_c = pos.astype(jnp.int32).T
    act_c = act.astype(jnp.int32).T
    slot_m = jax.lax.broadcasted_iota(jnp.int32, (E, E), 1)
    e_m = jax.lax.broadcasted_iota(jnp.int32, (E, E), 0)
    take = (pos_c == slot_m) & (act_c == 1)
    order = jnp.sum(jnp.where(take, e_m, 0), axis=0, keepdims=True)
    ncol = jnp.full((E, 1), nactf, jnp.float32).astype(jnp.int32)
    tbl_ref[...] = jnp.concatenate(
        [order.T, ncol, jnp.zeros((E, 6), jnp.int32)], axis=1)

    nact = tbl_ref[0, 1]
    out_ref[...] = jnp.zeros_like(out_ref)

    def issue(slot, i):
        e = tbl_ref[i, 0]
        pltpu.make_async_copy(wup_hbm.at[e], ubuf.at[slot],
                              sem_u.at[slot]).start()
        pltpu.make_async_copy(wdn_hbm.at[e], dbuf.at[slot],
                              sem_d.at[slot]).start()

    issue(0, 0)

    def step(i, carry):
        slot = jax.lax.rem(i, 2)
        nxt = jax.lax.rem(i + 1, 2)

        @pl.when(i + 1 < nact)
        def _prefetch():
            issue(nxt, i + 1)

        e = tbl_ref[i, 0]
        pltpu.make_async_copy(wup_hbm.at[e], ubuf.at[slot],
                              sem_u.at[slot]).wait()
        pltpu.make_async_copy(wdn_hbm.at[e], dbuf.at[slot],
                              sem_d.at[slot]).wait()
        g = gates_ref[e, :].reshape(-1, 1)
        h = jnp.dot(x, ubuf[slot].T, preferred_element_type=jnp.float32)
        h = h + bu_ref[e, :]
        h = 0.5 * h * (1.0 + jax.lax.erf(h * 0.7071067811865476))
        h = h * g
        out_ref[...] += jnp.dot(h, dbuf[slot].T,
                                preferred_element_type=jnp.float32)
        out_ref[...] += g * bd_ref[e, :]
        return carry

    jax.lax.fori_loop(0, nact, step, 0)


def kernel(x, expert_weights_up, expert_weights_down, expert_biases_up,
           expert_biases_down, router_weight):
    B, S, D = x.shape
    E, U, _ = expert_weights_up.shape
    x2d = x.reshape(B * S, D)

    out = pl.pallas_call(
        _moe_body,
        in_specs=[
            pl.BlockSpec((B * S, D), lambda: (0, 0)),
            pl.BlockSpec((E, D), lambda: (0, 0)),
            pl.BlockSpec((E, U), lambda: (0, 0)),
            pl.BlockSpec((E, D), lambda: (0, 0)),
            pl.BlockSpec(memory_space=pltpu.MemorySpace.HBM),
            pl.BlockSpec(memory_space=pltpu.MemorySpace.HBM),
        ],
        out_specs=pl.BlockSpec((B * S, D), lambda: (0, 0)),
        out_shape=jax.ShapeDtypeStruct((B * S, D), jnp.float32),
        scratch_shapes=[
            pltpu.VMEM((E, B * S), jnp.float32),
            pltpu.VMEM((E, 8), jnp.int32),
            pltpu.VMEM((2, U, D), jnp.float32),
            pltpu.VMEM((2, D, U), jnp.float32),
            pltpu.SemaphoreType.DMA((2,)),
            pltpu.SemaphoreType.DMA((2,)),
        ],
    )(x2d, router_weight, expert_biases_up, expert_biases_down,
      expert_weights_up, expert_weights_down)
    return out.reshape(B, S, D)


# speedup vs baseline: 1.1319x; 1.0085x over previous
import jax
import jax.numpy as jnp
from jax.experimental import pallas as pl
from jax.experimental.pallas import tpu as pltpu


def _moe_body(x_ref, rw_ref, bu_ref, bd_ref, wup_hbm, wdn_hbm, out_ref,
              gates_ref, tbl_ref, ubuf, dbuf, sem_u, sem_d):
    x = x_ref[...]
    rw = rw_ref[...]
    B, E = x.shape[0], rw.shape[0]

    logits = jnp.dot(x, rw.T, preferred_element_type=jnp.float32)
    col = jax.lax.broadcasted_iota(jnp.int32, (B, E), 1)
    m1 = jnp.max(logits, axis=1, keepdims=True)
    i1 = jnp.min(jnp.where(logits == m1, col, E), axis=1, keepdims=True)
    masked = jnp.where(col == i1, -jnp.inf, logits)
    m2 = jnp.max(masked, axis=1, keepdims=True)
    i2 = jnp.min(jnp.where(masked == m2, col, E), axis=1, keepdims=True)
    t = jnp.exp(m2 - m1)
    w1 = 1.0 / (1.0 + t)
    w2 = t / (1.0 + t)
    sel1 = (col == i1)
    sel2 = (col == i2)
    gates_ref[...] = (w1 * sel1 + w2 * sel2).T

    act = jnp.max((sel1 | sel2).astype(jnp.float32), axis=0, keepdims=True)
    lt = (jax.lax.broadcasted_iota(jnp.int32, (E, E), 0) <
          jax.lax.broadcasted_iota(jnp.int32, (E, E), 1)).astype(jnp.float32)
    pos = jnp.dot(act, lt, preferred_element_type=jnp.float32)
    nactf = jnp.sum(act, dtype=jnp.float32)
    pos_c = pos.astype(jnp.int32).T
    act_c = act.astype(jnp.int32).T
    slot_m = jax.lax.broadcasted_iota(jnp.int32, (E, E), 1)
    e_m = jax.lax.broadcasted_iota(jnp.int32, (E, E), 0)
    take = (pos_c == slot_m) & (act_c == 1)
    order = jnp.sum(jnp.where(take, e_m, 0), axis=0, keepdims=True)
    ncol = jnp.full((E, 1), nactf, jnp.float32).astype(jnp.int32)
    tbl_ref[...] = jnp.concatenate(
        [order.T, ncol, jnp.zeros((E, 6), jnp.int32)], axis=1)

    nact = tbl_ref[0, 1]
    out_ref[...] = jnp.zeros_like(out_ref)
    U = ubuf.shape[1] * 2
    CH = U // 2
    DH = x.shape[1] // 2

    def issue(half, i):
        e = tbl_ref[i, 0]
        for c in range(2):
            pltpu.make_async_copy(wup_hbm.at[e, pl.ds(c * CH, CH), :],
                                  ubuf.at[half * 2 + c],
                                  sem_u.at[half * 2 + c]).start()
            pltpu.make_async_copy(wdn_hbm.at[e, pl.ds(c * DH, DH), :],
                                  dbuf.at[half * 2 + c],
                                  sem_d.at[half * 2 + c]).start()

    issue(0, 0)

    def step(i, carry):
        half = jax.lax.rem(i, 2)

        @pl.when(i + 1 < nact)
        def _prefetch():
            issue(jax.lax.rem(i + 1, 2), i + 1)

        e = tbl_ref[i, 0]
        g = gates_ref[e, :].reshape(-1, 1)
        hs = []
        for c in range(2):
            pltpu.make_async_copy(wup_hbm.at[e, pl.ds(c * CH, CH), :],
                                  ubuf.at[half * 2 + c],
                                  sem_u.at[half * 2 + c]).wait()
            hs.append(jnp.dot(x, ubuf[half * 2 + c].T,
                              preferred_element_type=jnp.float32))
        h = jnp.concatenate(hs, axis=1)
        h = h + bu_ref[e, :]
        h = 0.5 * h * (1.0 + jax.lax.erf(h * 0.7071067811865476))
        h = h * g
        for c in range(2):
            pltpu.make_async_copy(wdn_hbm.at[e, pl.ds(c * DH, DH), :],
                                  dbuf.at[half * 2 + c],
                                  sem_d.at[half * 2 + c]).wait()
            out_ref[:, c * DH:(c + 1) * DH] += jnp.dot(
                h, dbuf[half * 2 + c].T, preferred_element_type=jnp.float32)
        out_ref[...] += g * bd_ref[e, :]
        return carry

    jax.lax.fori_loop(0, nact, step, 0)


def kernel(x, expert_weights_up, expert_weights_down, expert_biases_up,
           expert_biases_down, router_weight):
    B, S, D = x.shape
    E, U, _ = expert_weights_up.shape
    x2d = x.reshape(B * S, D)

    out = pl.pallas_call(
        _moe_body,
        in_specs=[
            pl.BlockSpec((B * S, D), lambda: (0, 0)),
            pl.BlockSpec((E, D), lambda: (0, 0)),
            pl.BlockSpec((E, U), lambda: (0, 0)),
            pl.BlockSpec((E, D), lambda: (0, 0)),
            pl.BlockSpec(memory_space=pltpu.MemorySpace.HBM),
            pl.BlockSpec(memory_space=pltpu.MemorySpace.HBM),
        ],
        out_specs=pl.BlockSpec((B * S, D), lambda: (0, 0)),
        out_shape=jax.ShapeDtypeStruct((B * S, D), jnp.float32),
        scratch_shapes=[
            pltpu.VMEM((E, B * S), jnp.float32),
            pltpu.VMEM((E, 8), jnp.int32),
            pltpu.VMEM((4, U // 2, D), jnp.float32),
            pltpu.VMEM((4, D // 2, U), jnp.float32),
            pltpu.SemaphoreType.DMA((4,)),
            pltpu.SemaphoreType.DMA((4,)),
        ],
    )(x2d, router_weight, expert_biases_up, expert_biases_down,
      expert_weights_up, expert_weights_down)
    return out.reshape(B, S, D)
